# Initial kernel scaffold; baseline (speedup 1.0000x reference)
import jax
import jax.numpy as jnp
from jax.experimental import pallas as pl


def kernel(x_num, x_cat, tables, W1, b1, g1, be1, m1, v1, W2, b2, g2, be2, m2, v2, W3, b3):
    raise NotImplementedError("write your pallas kernel here")


# speedup vs baseline: 7.6968x; 7.6968x over previous
import functools

import jax
import jax.numpy as jnp
from jax import lax
from jax.experimental import pallas as pl
from jax.experimental.pallas import tpu as pltpu
from jax.experimental.pallas import tpu_sc as plsc

B = 16384
NNUM = 13
NF = 26
V = 100000
ED = 16
H = 128
EPS = 1e-5

ROWS = B * NF
NC, NS = 2, 16
NW = NC * NS
ROWS_PER_W = ROWS // NW
CHUNK = 1024
NCHUNK = ROWS_PER_W // CHUNK

@functools.cache
def _sc_gather_fn():
    mesh = plsc.VectorSubcoreMesh(core_axis_name="c", subcore_axis_name="s")

    @functools.partial(
        pl.kernel,
        out_type=jax.ShapeDtypeStruct((ROWS, ED), jnp.float32),
        mesh=mesh,
        scratch_types=[
            pltpu.VMEM((CHUNK,), jnp.int32),
            pltpu.VMEM((CHUNK, ED), jnp.float32),
            pltpu.SemaphoreType.DMA,
        ],
        compiler_params=pltpu.CompilerParams(use_tc_tiling_on_sc=False),
    )
    def _sc_gather(tab_hbm, idx_hbm, out_hbm, idx_v, rows_v, sem):
        wid = lax.axis_index("s") * NC + lax.axis_index("c")
        base = wid * ROWS_PER_W

        def body(i, carry):
            off = base + i * CHUNK
            pltpu.sync_copy(idx_hbm.at[pl.ds(off, CHUNK)], idx_v)
            pltpu.async_copy(tab_hbm.at[idx_v], rows_v, sem).wait()
            pltpu.sync_copy(rows_v, out_hbm.at[pl.ds(off, CHUNK)])
            return carry

        lax.fori_loop(0, NCHUNK, body, 0)

    return _sc_gather


def _mlp_body(xn_ref, emb_ref, w1n_ref, w1e_ref, b1_ref, g1_ref, be1_ref,
              m1_ref, v1_ref, w2_ref, b2_ref, g2_ref, be2_ref, m2_ref,
              v2_ref, w3_ref, b3_ref, out_ref):
    h = jnp.dot(xn_ref[...], w1n_ref[...], preferred_element_type=jnp.float32)
    h = h + jnp.dot(emb_ref[...], w1e_ref[...],
                    preferred_element_type=jnp.float32)
    h = jnp.maximum(h + b1_ref[...], 0.0)
    h = (h - m1_ref[...]) / jnp.sqrt(v1_ref[...] + EPS) * g1_ref[...] \
        + be1_ref[...]
    h = jnp.dot(h, w2_ref[...], preferred_element_type=jnp.float32)
    h = jnp.maximum(h + b2_ref[...], 0.0)
    h = (h - m2_ref[...]) / jnp.sqrt(v2_ref[...] + EPS) * g2_ref[...] \
        + be2_ref[...]
    out_ref[...] = jnp.dot(h, w3_ref[...],
                           preferred_element_type=jnp.float32) + b3_ref[...]


BM = 1024


def _mlp(x_num, emb, w1n, w1e, b1, g1, be1, m1, v1, w2, b2, g2, be2, m2, v2,
         w3, b3):
    n_blocks = B // BM
    row_block = lambda i: (i, 0)
    const = pl.BlockSpec(None, lambda i: None)
    full = lambda shape: pl.BlockSpec(shape, lambda i: (0, 0))
    return pl.pallas_call(
        _mlp_body,
        grid=(n_blocks,),
        in_specs=[
            pl.BlockSpec((BM, NNUM), row_block),
            pl.BlockSpec((BM, NF * ED), row_block),
            full((NNUM, H)),
            full((NF * ED, H)),
            full((1, H)), full((1, H)), full((1, H)), full((1, H)),
            full((1, H)),
            full((H, H // 2)),
            full((1, H // 2)), full((1, H // 2)), full((1, H // 2)),
            full((1, H // 2)), full((1, H // 2)),
            full((H // 2, 1)),
            full((1, 1)),
        ],
        out_specs=pl.BlockSpec((BM, 1), row_block),
        out_shape=jax.ShapeDtypeStruct((B, 1), jnp.float32),
    )(x_num, emb, w1n, w1e, b1, g1, be1, m1, v1, w2, b2, g2, be2, m2, v2,
      w3, b3)


def kernel(x_num, x_cat, tables, W1, b1, g1, be1, m1, v1, W2, b2, g2, be2,
           m2, v2, W3, b3):
    flat_tab = tables.reshape(NF * V, ED)
    idx = (x_cat + jnp.arange(NF, dtype=jnp.int32)[None, :] * V).reshape(-1)
    emb = _sc_gather_fn()(flat_tab, idx).reshape(B, NF * ED)
    out = _mlp(x_num, emb,
               W1[:NNUM], W1[NNUM:],
               b1.reshape(1, H), g1.reshape(1, H), be1.reshape(1, H),
               m1.reshape(1, H), v1.reshape(1, H),
               W2,
               b2.reshape(1, H // 2), g2.reshape(1, H // 2),
               be2.reshape(1, H // 2), m2.reshape(1, H // 2),
               v2.reshape(1, H // 2),
               W3, b3.reshape(1, 1))
    return out


# speedup vs baseline: 7.7877x; 1.0118x over previous
import functools

import jax
import jax.numpy as jnp
from jax import lax
from jax.experimental import pallas as pl
from jax.experimental.pallas import tpu as pltpu
from jax.experimental.pallas import tpu_sc as plsc

B = 16384
NNUM = 13
NF = 26
V = 100000
ED = 16
H = 128
EPS = 1e-5

ROWS = B * NF
NC, NS = 2, 16
NW = NC * NS
ROWS_PER_W = ROWS // NW
CHUNK = 1664
NCHUNK = ROWS_PER_W // CHUNK


@functools.cache
def _sc_gather_fn():
    mesh = plsc.VectorSubcoreMesh(core_axis_name="c", subcore_axis_name="s")

    @functools.partial(
        pl.kernel,
        out_type=jax.ShapeDtypeStruct((ROWS, ED), jnp.float32),
        mesh=mesh,
        scratch_types=[
            pltpu.VMEM((ROWS_PER_W,), jnp.int32),
            pltpu.VMEM((2, CHUNK, ED), jnp.float32),
            pltpu.SemaphoreType.DMA,
            pltpu.SemaphoreType.DMA,
            pltpu.SemaphoreType.DMA,
        ],
        compiler_params=pltpu.CompilerParams(use_tc_tiling_on_sc=False),
    )
    def _sc_gather(tab_hbm, idx_hbm, out_hbm, idx_v, rows_v, gsem, ssem,
                   isem):
        wid = lax.axis_index("s") * NC + lax.axis_index("c")
        base = wid * ROWS_PER_W
        pltpu.async_copy(idx_hbm.at[pl.ds(base, ROWS_PER_W)], idx_v,
                         isem).wait()

        def gather(i, buf):
            return pltpu.async_copy(
                tab_hbm.at[idx_v.at[pl.ds(i * CHUNK, CHUNK)]], buf, gsem)

        def store(i, buf):
            return pltpu.async_copy(
                buf, out_hbm.at[pl.ds(base + i * CHUNK, CHUNK)], ssem)

        gather(0, rows_v.at[0])
        for i in range(NCHUNK):
            cur = rows_v.at[i % 2]
            nxt = rows_v.at[(i + 1) % 2]
            pltpu.make_async_copy(
                tab_hbm.at[idx_v.at[pl.ds(i * CHUNK, CHUNK)]], cur,
                gsem).wait()
            if i > 0:
                pltpu.make_async_copy(
                    rows_v.at[(i - 1) % 2],
                    out_hbm.at[pl.ds(base + (i - 1) * CHUNK, CHUNK)],
                    ssem).wait()
            if i + 1 < NCHUNK:
                gather(i + 1, nxt)
            store(i, cur)
        pltpu.make_async_copy(
            rows_v.at[(NCHUNK - 1) % 2],
            out_hbm.at[pl.ds(base + (NCHUNK - 1) * CHUNK, CHUNK)],
            ssem).wait()

    return _sc_gather


def _mlp_body(xn_ref, emb_ref, w1n_ref, w1e_ref, b1_ref, g1_ref, be1_ref,
              m1_ref, v1_ref, w2_ref, b2_ref, g2_ref, be2_ref, m2_ref,
              v2_ref, w3_ref, b3_ref, out_ref):
    h = jnp.dot(xn_ref[...], w1n_ref[...], preferred_element_type=jnp.float32)
    h = h + jnp.dot(emb_ref[...], w1e_ref[...],
                    preferred_element_type=jnp.float32)
    h = jnp.maximum(h + b1_ref[...], 0.0)
    h = (h - m1_ref[...]) / jnp.sqrt(v1_ref[...] + EPS) * g1_ref[...] \
        + be1_ref[...]
    h = jnp.dot(h, w2_ref[...], preferred_element_type=jnp.float32)
    h = jnp.maximum(h + b2_ref[...], 0.0)
    h = (h - m2_ref[...]) / jnp.sqrt(v2_ref[...] + EPS) * g2_ref[...] \
        + be2_ref[...]
    out_ref[...] = jnp.dot(h, w3_ref[...],
                           preferred_element_type=jnp.float32) + b3_ref[...]


BM = 1024


def _mlp(x_num, emb, w1n, w1e, b1, g1, be1, m1, v1, w2, b2, g2, be2, m2, v2,
         w3, b3):
    n_blocks = B // BM
    row_block = lambda i: (i, 0)
    full = lambda shape: pl.BlockSpec(shape, lambda i: (0, 0))
    return pl.pallas_call(
        _mlp_body,
        grid=(n_blocks,),
        in_specs=[
            pl.BlockSpec((BM, NNUM), row_block),
            pl.BlockSpec((BM, NF * ED), row_block),
            full((NNUM, H)),
            full((NF * ED, H)),
            full((1, H)), full((1, H)), full((1, H)), full((1, H)),
            full((1, H)),
            full((H, H // 2)),
            full((1, H // 2)), full((1, H // 2)), full((1, H // 2)),
            full((1, H // 2)), full((1, H // 2)),
            full((H // 2, 1)),
            full((1, 1)),
        ],
        out_specs=pl.BlockSpec((BM, 1), row_block),
        out_shape=jax.ShapeDtypeStruct((B, 1), jnp.float32),
    )(x_num, emb, w1n, w1e, b1, g1, be1, m1, v1, w2, b2, g2, be2, m2, v2,
      w3, b3)


def kernel(x_num, x_cat, tables, W1, b1, g1, be1, m1, v1, W2, b2, g2, be2,
           m2, v2, W3, b3):
    flat_tab = tables.reshape(NF * V, ED)
    idx = (x_cat + jnp.arange(NF, dtype=jnp.int32)[None, :] * V).reshape(-1)
    emb = _sc_gather_fn()(flat_tab, idx).reshape(B, NF * ED)
    out = _mlp(x_num, emb,
               W1[:NNUM], W1[NNUM:],
               b1.reshape(1, H), g1.reshape(1, H), be1.reshape(1, H),
               m1.reshape(1, H), v1.reshape(1, H),
               W2,
               b2.reshape(1, H // 2), g2.reshape(1, H // 2),
               be2.reshape(1, H // 2), m2.reshape(1, H // 2),
               v2.reshape(1, H // 2),
               W3, b3.reshape(1, 1))
    return out
